# baseline (speedup 1.0000x reference)
import functools

import jax
import jax.numpy as jnp
from jax import lax
from jax.experimental import pallas as pl
from jax.experimental.pallas import tpu as pltpu
from jax.experimental.pallas import tpu_sc as plsc

N = 10000
E = 320000
D = 128
G = 100
NPG = 100
K = 30
TL = 97

NC, NS = 2, 16
NW = NC * NS
CHUNK = 128
RPW = 313
NCHUNK = 86
EPW = NCHUNK * CHUNK
N_PAD = 10240
RPS = N_PAD // NS


def _edge_agg_body(with_deg, d, h_hbm, src_hbm, dst_hbm, zeros_hbm, zeros1_hbm,
                   *refs):
    dacc = None
    if with_deg:
        agg_out, deg_out, src_v, dst_v, rows_v, ones_v, accum, dacc, sem = refs
    else:
        agg_out, src_v, dst_v, rows_v, accum, sem = refs
    c = lax.axis_index("c")
    s = lax.axis_index("s")
    wid = s * NC + c

    if True:
        pltpu.sync_copy(zeros_hbm.at[pl.ds(s * RPS, RPS)],
                        accum.at[pl.ds(s * RPS, RPS)])
        if with_deg:
            pltpu.sync_copy(zeros1_hbm.at[pl.ds(s * RPS, RPS)],
                            dacc.at[pl.ds(s * RPS, RPS)])
            for t in range(CHUNK // 16):
                ones_v[pl.ds(t * 16, 16)] = jnp.full((16,), 1.0, jnp.float32)
        plsc.subcore_barrier()

        pltpu.sync_copy(src_hbm.at[wid], src_v)
        pltpu.sync_copy(dst_hbm.at[wid], dst_v)

        def chunk_body(j, carry):
            pltpu.async_copy(h_hbm.at[src_v.at[j]], rows_v, sem).wait()
            pltpu.sync_copy(rows_v, accum.at[dst_v.at[j]], add=True)
            if with_deg:
                pltpu.sync_copy(ones_v, dacc.at[dst_v.at[j]], add=True)
            return carry

        lax.fori_loop(0, NCHUNK, chunk_body, 0)
        plsc.subcore_barrier()

        pltpu.sync_copy(accum.at[pl.ds(s * RPS, RPS)],
                        agg_out.at[pl.ds(c * N_PAD + s * RPS, RPS)])
        if with_deg:
            pltpu.sync_copy(dacc.at[pl.ds(s * RPS, RPS)],
                            deg_out.at[pl.ds(c * N_PAD + s * RPS, RPS)])


def _edge_agg(h, src_r, dst_r, zeros_pad, zeros1, with_deg):
    d = h.shape[1]
    out_type = [jax.ShapeDtypeStruct((NC * N_PAD, d), jnp.float32)]
    scratch = [
        pltpu.VMEM((NCHUNK, CHUNK), jnp.int32),
        pltpu.VMEM((NCHUNK, CHUNK), jnp.int32),
        pltpu.VMEM((CHUNK, d), jnp.float32),
    ]
    if with_deg:
        out_type.append(jax.ShapeDtypeStruct((NC * N_PAD,), jnp.float32))
        scratch.append(pltpu.VMEM((CHUNK,), jnp.float32))
    scratch.append(pltpu.VMEM_SHARED((N_PAD, d), jnp.float32))
    if with_deg:
        scratch.append(pltpu.VMEM_SHARED((N_PAD,), jnp.float32))
    scratch.append(pltpu.SemaphoreType.DMA)
    mesh = plsc.VectorSubcoreMesh(core_axis_name="c", subcore_axis_name="s")
    fn = pl.kernel(
        functools.partial(_edge_agg_body, with_deg, d),
        compiler_params=pltpu.CompilerParams(use_tc_tiling_on_sc=False),
        out_type=tuple(out_type),
        mesh=mesh,
        scratch_types=tuple(scratch),
    )
    return fn(h, src_r, dst_r, zeros_pad, zeros1)


def _combine0_body(p0, p1, h, w, b, d0, d1, h_out, degs_out):
    degs = d0[...] + d1[...] + 1.0
    degs_out[...] = degs
    agg = p0[...] + p1[...] + h[...]
    lin = jnp.dot(agg, w[...]) + b[...]
    h_out[...] = jnp.tanh(lin / degs)


def _combine_body(p0, p1, h, w, b, degs, h_out):
    agg = p0[...] + p1[...] + h[...]
    lin = jnp.dot(agg, w[...]) + b[...]
    h_out[...] = jnp.tanh(lin / degs[...])


def _combine0(p0, p1, h, w, b, d0, d1):
    return pl.pallas_call(
        _combine0_body,
        out_shape=(
            jax.ShapeDtypeStruct((N, w.shape[1]), jnp.float32),
            jax.ShapeDtypeStruct((N, 1), jnp.float32),
        ),
    )(p0, p1, h, w, b, d0, d1)


def _combine(p0, p1, h, w, b, degs):
    return pl.pallas_call(
        _combine_body,
        out_shape=jax.ShapeDtypeStruct((N, w.shape[1]), jnp.float32),
    )(p0, p1, h, w, b, degs)


def kernel(node_feat, edge_index, W0, b0, W1, b1, W2, b2, W3, b3, Wc1, bc1, Wc2, bc2, Wd, bd):
    src = edge_index[0]
    dst = edge_index[1]
    bucket = dst // RPW
    order = jnp.argsort(bucket, stable=True)
    src_o = src[order]
    dst_o = dst[order]
    bucket_o = bucket[order]
    counts = jnp.bincount(bucket, length=NW)
    offs = jnp.concatenate([jnp.zeros((1,), counts.dtype), jnp.cumsum(counts)[:-1]])
    rank = jnp.arange(E, dtype=jnp.int32) - offs[bucket_o].astype(jnp.int32)
    pos = bucket_o.astype(jnp.int32) * EPW + rank
    fill_src = (jnp.arange(NW * EPW, dtype=jnp.int32) * 997) % N
    src_r = fill_src.at[pos].set(src_o, unique_indices=True).reshape(NW, NCHUNK, CHUNK)
    dst_r = jnp.full((NW * EPW,), N, jnp.int32).at[pos].set(dst_o, unique_indices=True).reshape(NW, NCHUNK, CHUNK)
    zeros128 = jnp.zeros((N_PAD, D), jnp.float32)
    zeros1 = jnp.zeros((N_PAD,), jnp.float32)

    h = node_feat
    degs = None
    cats = []
    for i, (W, b) in enumerate(((W0, b0), (W1, b1), (W2, b2), (W3, b3))):
        zp = zeros128[:, : h.shape[1]]
        if i == 0:
            aggp, degp = _edge_agg(h, src_r, dst_r, zp, zeros1, True)
            p0, p1 = aggp[:N], aggp[N_PAD : N_PAD + N]
            d0, d1 = degp[:N, None], degp[N_PAD : N_PAD + N, None]
            h, degs = _combine0(p0, p1, h, W, b, d0, d1)
        else:
            (aggp,) = _edge_agg(h, src_r, dst_r, zp, zeros1, False)
            p0, p1 = aggp[:N], aggp[N_PAD : N_PAD + N]
            h = _combine(p0, p1, h, W, b, degs)
        cats.append(h)

    cm = jnp.concatenate(cats, axis=1)
    sort_channel = cm[:, -1].reshape(G, NPG)
    _, topk_idx = jax.lax.top_k(sort_channel, K)
    feats = cm.reshape(G, NPG, TL)
    pooled = jnp.take_along_axis(feats, topk_idx[:, :, None], axis=1)
    x = pooled.reshape(G, 1, K * TL)
    dn = ('NCH', 'OIH', 'NCH')
    y = jax.lax.conv_general_dilated(x, Wc1, (TL,), 'VALID', dimension_numbers=dn) + bc1[None, :, None]
    y = jax.nn.relu(y)
    y = jax.lax.reduce_window(y, -jnp.inf, jax.lax.max, (1, 1, 2), (1, 1, 2), 'VALID')
    y = jax.lax.conv_general_dilated(y, Wc2, (1,), 'VALID', dimension_numbers=dn) + bc2[None, :, None]
    y = jax.nn.relu(y)
    flat = y.reshape(G, -1)
    out = jax.nn.relu(flat @ Wd + bd)
    return jax.nn.relu(out)


# speedup vs baseline: 1.1326x; 1.1326x over previous
import functools

import jax
import jax.numpy as jnp
from jax import lax
from jax.experimental import pallas as pl
from jax.experimental.pallas import tpu as pltpu
from jax.experimental.pallas import tpu_sc as plsc

N = 10000
E = 320000
D = 128
G = 100
NPG = 100
K = 30
TL = 97

NC, NS = 2, 16
NW = NC * NS
CHUNK = 128
RPW = 313
NCHUNK = 86
EPW = NCHUNK * CHUNK
N_PAD = 10240
RPS = N_PAD // NS


def _edge_agg_body(with_deg, d, h_hbm, src_hbm, dst_hbm, zeros_hbm, zeros1_hbm,
                   *refs):
    dacc = None
    if with_deg:
        agg_out, deg_out, src_v, dst_v, rows_v, ones_v, accum, dacc, sem = refs
    else:
        agg_out, src_v, dst_v, rows_v, accum, sem = refs
    c = lax.axis_index("c")
    s = lax.axis_index("s")
    wid = s * NC + c

    if True:
        pltpu.sync_copy(zeros_hbm.at[pl.ds(s * RPS, RPS)],
                        accum.at[pl.ds(s * RPS, RPS)])
        if with_deg:
            pltpu.sync_copy(zeros1_hbm.at[pl.ds(s * RPS, RPS)],
                            dacc.at[pl.ds(s * RPS, RPS)])
            for t in range(CHUNK // 16):
                ones_v[pl.ds(t * 16, 16)] = jnp.full((16,), 1.0, jnp.float32)
        plsc.subcore_barrier()

        pltpu.sync_copy(src_hbm.at[wid], src_v)
        pltpu.sync_copy(dst_hbm.at[wid], dst_v)

        def chunk_body(j, carry):
            pltpu.async_copy(h_hbm.at[src_v.at[j]], rows_v, sem).wait()
            pltpu.sync_copy(rows_v, accum.at[dst_v.at[j]], add=True)
            if with_deg:
                pltpu.sync_copy(ones_v, dacc.at[dst_v.at[j]], add=True)
            return carry

        lax.fori_loop(0, NCHUNK, chunk_body, 0)
        plsc.subcore_barrier()

        pltpu.sync_copy(accum.at[pl.ds(s * RPS, RPS)],
                        agg_out.at[pl.ds(c * N_PAD + s * RPS, RPS)])
        if with_deg:
            pltpu.sync_copy(dacc.at[pl.ds(s * RPS, RPS)],
                            deg_out.at[pl.ds(c * N_PAD + s * RPS, RPS)])


def _edge_agg(h, src_r, dst_r, zeros_pad, zeros1, with_deg):
    d = h.shape[1]
    out_type = [jax.ShapeDtypeStruct((NC * N_PAD, d), jnp.float32)]
    scratch = [
        pltpu.VMEM((NCHUNK, CHUNK), jnp.int32),
        pltpu.VMEM((NCHUNK, CHUNK), jnp.int32),
        pltpu.VMEM((CHUNK, d), jnp.float32),
    ]
    if with_deg:
        out_type.append(jax.ShapeDtypeStruct((NC * N_PAD,), jnp.float32))
        scratch.append(pltpu.VMEM((CHUNK,), jnp.float32))
    scratch.append(pltpu.VMEM_SHARED((N_PAD, d), jnp.float32))
    if with_deg:
        scratch.append(pltpu.VMEM_SHARED((N_PAD,), jnp.float32))
    scratch.append(pltpu.SemaphoreType.DMA)
    mesh = plsc.VectorSubcoreMesh(core_axis_name="c", subcore_axis_name="s")
    fn = pl.kernel(
        functools.partial(_edge_agg_body, with_deg, d),
        compiler_params=pltpu.CompilerParams(use_tc_tiling_on_sc=False),
        out_type=tuple(out_type),
        mesh=mesh,
        scratch_types=tuple(scratch),
    )
    return fn(h, src_r, dst_r, zeros_pad, zeros1)


def _combine0_body(p0, p1, h, w, b, d0, d1, h_out, degs_out):
    degs = d0[...] + d1[...] + 1.0
    degs_out[...] = degs
    agg = p0[...] + p1[...] + h[...]
    lin = jnp.dot(agg, w[...]) + b[...]
    h_out[...] = jnp.tanh(lin / degs)


def _combine_body(p0, p1, h, w, b, degs, h_out):
    agg = p0[...] + p1[...] + h[...]
    lin = jnp.dot(agg, w[...]) + b[...]
    h_out[...] = jnp.tanh(lin / degs[...])


def _combine0(p0, p1, h, w, b, d0, d1):
    return pl.pallas_call(
        _combine0_body,
        out_shape=(
            jax.ShapeDtypeStruct((N, w.shape[1]), jnp.float32),
            jax.ShapeDtypeStruct((N, 1), jnp.float32),
        ),
    )(p0, p1, h, w, b, d0, d1)


def _combine(p0, p1, h, w, b, degs):
    return pl.pallas_call(
        _combine_body,
        out_shape=jax.ShapeDtypeStruct((N, w.shape[1]), jnp.float32),
    )(p0, p1, h, w, b, degs)


def kernel(node_feat, edge_index, W0, b0, W1, b1, W2, b2, W3, b3, Wc1, bc1, Wc2, bc2, Wd, bd):
    src = edge_index[0]
    dst = edge_index[1]
    bucket = dst // RPW
    onehot = (bucket[:, None] == jnp.arange(NW, dtype=jnp.int32)[None, :]).astype(jnp.int32)
    rank = jnp.take_along_axis(jnp.cumsum(onehot, axis=0), bucket[:, None], axis=1)[:, 0] - 1
    pos = bucket * EPW + rank
    fill_src = (jnp.arange(NW * EPW, dtype=jnp.int32) * 997) % N
    src_r = fill_src.at[pos].set(src, unique_indices=True).reshape(NW, NCHUNK, CHUNK)
    dst_r = jnp.full((NW * EPW,), N, jnp.int32).at[pos].set(dst, unique_indices=True).reshape(NW, NCHUNK, CHUNK)
    zeros128 = jnp.zeros((N_PAD, D), jnp.float32)
    zeros1 = jnp.zeros((N_PAD,), jnp.float32)

    h = node_feat
    degs = None
    cats = []
    for i, (W, b) in enumerate(((W0, b0), (W1, b1), (W2, b2), (W3, b3))):
        zp = zeros128[:, : h.shape[1]]
        if i == 0:
            aggp, degp = _edge_agg(h, src_r, dst_r, zp, zeros1, True)
            p0, p1 = aggp[:N], aggp[N_PAD : N_PAD + N]
            d0, d1 = degp[:N, None], degp[N_PAD : N_PAD + N, None]
            h, degs = _combine0(p0, p1, h, W, b, d0, d1)
        else:
            (aggp,) = _edge_agg(h, src_r, dst_r, zp, zeros1, False)
            p0, p1 = aggp[:N], aggp[N_PAD : N_PAD + N]
            h = _combine(p0, p1, h, W, b, degs)
        cats.append(h)

    cm = jnp.concatenate(cats, axis=1)
    sort_channel = cm[:, -1].reshape(G, NPG)
    _, topk_idx = jax.lax.top_k(sort_channel, K)
    feats = cm.reshape(G, NPG, TL)
    pooled = jnp.take_along_axis(feats, topk_idx[:, :, None], axis=1)
    x = pooled.reshape(G, 1, K * TL)
    dn = ('NCH', 'OIH', 'NCH')
    y = jax.lax.conv_general_dilated(x, Wc1, (TL,), 'VALID', dimension_numbers=dn) + bc1[None, :, None]
    y = jax.nn.relu(y)
    y = jax.lax.reduce_window(y, -jnp.inf, jax.lax.max, (1, 1, 2), (1, 1, 2), 'VALID')
    y = jax.lax.conv_general_dilated(y, Wc2, (1,), 'VALID', dimension_numbers=dn) + bc2[None, :, None]
    y = jax.nn.relu(y)
    flat = y.reshape(G, -1)
    out = jax.nn.relu(flat @ Wd + bd)
    return jax.nn.relu(out)


# speedup vs baseline: 1.1357x; 1.0028x over previous
import functools

import jax
import jax.numpy as jnp
from jax import lax
from jax.experimental import pallas as pl
from jax.experimental.pallas import tpu as pltpu
from jax.experimental.pallas import tpu_sc as plsc

N = 10000
E = 320000
D = 128
G = 100
NPG = 100
K = 30
TL = 97

NC, NS = 2, 16
NW = NC * NS
CHUNK = 128
RPW = 313
NCHUNK = 86
EPW = NCHUNK * CHUNK
N_PAD = 10240
RPS = N_PAD // NS


def _edge_agg_body(with_deg, d, h_hbm, src_hbm, dst_hbm, zeros_hbm, zeros1_hbm,
                   *refs):
    dacc = None
    if with_deg:
        agg_out, deg_out, src_v, dst_v, rows_v, ones_v, accum, dacc, sem = refs
    else:
        agg_out, src_v, dst_v, rows_v, accum, sem = refs
    c = lax.axis_index("c")
    s = lax.axis_index("s")
    wid = s * NC + c

    if True:
        pltpu.sync_copy(zeros_hbm.at[pl.ds(s * RPS, RPS)],
                        accum.at[pl.ds(s * RPS, RPS)])
        if with_deg:
            pltpu.sync_copy(zeros1_hbm.at[pl.ds(s * RPS, RPS)],
                            dacc.at[pl.ds(s * RPS, RPS)])
            for t in range(CHUNK // 16):
                ones_v[pl.ds(t * 16, 16)] = jnp.full((16,), 1.0, jnp.float32)
        plsc.subcore_barrier()

        pltpu.sync_copy(src_hbm.at[wid], src_v)
        pltpu.sync_copy(dst_hbm.at[wid], dst_v)

        def chunk_body(j, carry):
            pltpu.async_copy(h_hbm.at[src_v.at[j]], rows_v, sem).wait()
            pltpu.sync_copy(rows_v, accum.at[dst_v.at[j]], add=True)
            if with_deg:
                pltpu.sync_copy(ones_v, dacc.at[dst_v.at[j]], add=True)
            return carry

        lax.fori_loop(0, NCHUNK, chunk_body, 0)
        plsc.subcore_barrier()

        pltpu.sync_copy(accum.at[pl.ds(s * RPS, RPS)],
                        agg_out.at[pl.ds(c * N_PAD + s * RPS, RPS)])
        if with_deg:
            pltpu.sync_copy(dacc.at[pl.ds(s * RPS, RPS)],
                            deg_out.at[pl.ds(c * N_PAD + s * RPS, RPS)])


def _edge_agg(h, src_r, dst_r, zeros_pad, zeros1, with_deg):
    d = h.shape[1]
    out_type = [jax.ShapeDtypeStruct((NC * N_PAD, d), jnp.float32)]
    scratch = [
        pltpu.VMEM((NCHUNK, CHUNK), jnp.int32),
        pltpu.VMEM((NCHUNK, CHUNK), jnp.int32),
        pltpu.VMEM((CHUNK, d), jnp.float32),
    ]
    if with_deg:
        out_type.append(jax.ShapeDtypeStruct((NC * N_PAD,), jnp.float32))
        scratch.append(pltpu.VMEM((CHUNK,), jnp.float32))
    scratch.append(pltpu.VMEM_SHARED((N_PAD, d), jnp.float32))
    if with_deg:
        scratch.append(pltpu.VMEM_SHARED((N_PAD,), jnp.float32))
    scratch.append(pltpu.SemaphoreType.DMA)
    mesh = plsc.VectorSubcoreMesh(core_axis_name="c", subcore_axis_name="s")
    fn = pl.kernel(
        functools.partial(_edge_agg_body, with_deg, d),
        compiler_params=pltpu.CompilerParams(use_tc_tiling_on_sc=False),
        out_type=tuple(out_type),
        mesh=mesh,
        scratch_types=tuple(scratch),
    )
    return fn(h, src_r, dst_r, zeros_pad, zeros1)


def _combine0_body(p0, p1, h, w, b, d0, d1, h_out, degs_out):
    degs = d0[...] + d1[...] + 1.0
    degs_out[...] = degs
    agg = p0[...] + p1[...] + h[...]
    lin = jnp.dot(agg, w[...]) + b[...]
    h_out[...] = jnp.tanh(lin / degs)


def _combine_body(p0, p1, h, w, b, degs, h_out):
    agg = p0[...] + p1[...] + h[...]
    lin = jnp.dot(agg, w[...]) + b[...]
    h_out[...] = jnp.tanh(lin / degs[...])


def _combine0(p0, p1, h, w, b, d0, d1):
    return pl.pallas_call(
        _combine0_body,
        out_shape=(
            jax.ShapeDtypeStruct((N, w.shape[1]), jnp.float32),
            jax.ShapeDtypeStruct((N, 1), jnp.float32),
        ),
    )(p0, p1, h, w, b, d0, d1)


def _combine(p0, p1, h, w, b, degs):
    return pl.pallas_call(
        _combine_body,
        out_shape=jax.ShapeDtypeStruct((N, w.shape[1]), jnp.float32),
    )(p0, p1, h, w, b, degs)


def _topk_body(h_ref, out_ref):
    v = h_ref[...]
    lane = jax.lax.broadcasted_iota(jnp.int32, v.shape, 1)
    gbase = jax.lax.broadcasted_iota(jnp.int32, (G, 1), 0) * NPG
    neg = jnp.float32(-jnp.inf)
    for k in range(K):
        m = jnp.max(v, axis=1, keepdims=True)
        is_max = v >= m
        idx = jnp.min(jnp.where(is_max, lane, jnp.int32(2**30)), axis=1, keepdims=True)
        out_ref[:, k : k + 1] = idx + gbase
        v = jnp.where(lane == idx, neg, v)


def _topk_indices(h4):
    h4p = jnp.pad(h4, ((0, 0), (0, 128 - NPG)), constant_values=-jnp.inf)
    return pl.pallas_call(
        _topk_body,
        out_shape=jax.ShapeDtypeStruct((G, K), jnp.int32),
    )(h4p)


def kernel(node_feat, edge_index, W0, b0, W1, b1, W2, b2, W3, b3, Wc1, bc1, Wc2, bc2, Wd, bd):
    src = edge_index[0]
    dst = edge_index[1]
    bucket = dst // RPW
    onehot = (bucket[:, None] == jnp.arange(NW, dtype=jnp.int32)[None, :]).astype(jnp.int32)
    rank = jnp.take_along_axis(jnp.cumsum(onehot, axis=0), bucket[:, None], axis=1)[:, 0] - 1
    pos = bucket * EPW + rank
    fill_src = (jnp.arange(NW * EPW, dtype=jnp.int32) * 997) % N
    src_r = fill_src.at[pos].set(src, unique_indices=True).reshape(NW, NCHUNK, CHUNK)
    dst_r = jnp.full((NW * EPW,), N, jnp.int32).at[pos].set(dst, unique_indices=True).reshape(NW, NCHUNK, CHUNK)
    zeros128 = jnp.zeros((N_PAD, D), jnp.float32)
    zeros1 = jnp.zeros((N_PAD,), jnp.float32)

    h = node_feat
    degs = None
    cats = []
    for i, (W, b) in enumerate(((W0, b0), (W1, b1), (W2, b2), (W3, b3))):
        zp = zeros128[:, : h.shape[1]]
        if i == 0:
            aggp, degp = _edge_agg(h, src_r, dst_r, zp, zeros1, True)
            p0, p1 = aggp[:N], aggp[N_PAD : N_PAD + N]
            d0, d1 = degp[:N, None], degp[N_PAD : N_PAD + N, None]
            h, degs = _combine0(p0, p1, h, W, b, d0, d1)
        else:
            (aggp,) = _edge_agg(h, src_r, dst_r, zp, zeros1, False)
            p0, p1 = aggp[:N], aggp[N_PAD : N_PAD + N]
            h = _combine(p0, p1, h, W, b, degs)
        cats.append(h)

    cm = jnp.concatenate(cats, axis=1)
    idxg = _topk_indices(cats[3][:, 0].reshape(G, NPG))
    pooled = cm[idxg.reshape(-1)].reshape(G, K, TL)
    x = pooled.reshape(G, 1, K * TL)
    dn = ('NCH', 'OIH', 'NCH')
    y = jax.lax.conv_general_dilated(x, Wc1, (TL,), 'VALID', dimension_numbers=dn) + bc1[None, :, None]
    y = jax.nn.relu(y)
    y = jax.lax.reduce_window(y, -jnp.inf, jax.lax.max, (1, 1, 2), (1, 1, 2), 'VALID')
    y = jax.lax.conv_general_dilated(y, Wc2, (1,), 'VALID', dimension_numbers=dn) + bc2[None, :, None]
    y = jax.nn.relu(y)
    flat = y.reshape(G, -1)
    out = jax.nn.relu(flat @ Wd + bd)
    return jax.nn.relu(out)


# speedup vs baseline: 1.1371x; 1.0012x over previous
import functools

import jax
import jax.numpy as jnp
from jax import lax
from jax.experimental import pallas as pl
from jax.experimental.pallas import tpu as pltpu
from jax.experimental.pallas import tpu_sc as plsc

N = 10000
E = 320000
D = 128
G = 100
NPG = 100
K = 30
TL = 97
C1, C2 = 16, 32
KW2 = 5
OUT = 128

NC, NS = 2, 16
NW = NC * NS
CHUNK = 128
RPW = 313
NCHUNK = 86
EPW = NCHUNK * CHUNK
N_PAD = 10240
RPS = N_PAD // NS


def _edge_agg_body(with_deg, d, h_hbm, src_hbm, dst_hbm, zeros_hbm, zeros1_hbm,
                   *refs):
    dacc = None
    if with_deg:
        agg_out, deg_out, src_v, dst_v, rows_v, ones_v, accum, dacc, sem = refs
    else:
        agg_out, src_v, dst_v, rows_v, accum, sem = refs
    c = lax.axis_index("c")
    s = lax.axis_index("s")
    wid = s * NC + c

    if True:
        pltpu.sync_copy(zeros_hbm.at[pl.ds(s * RPS, RPS)],
                        accum.at[pl.ds(s * RPS, RPS)])
        if with_deg:
            pltpu.sync_copy(zeros1_hbm.at[pl.ds(s * RPS, RPS)],
                            dacc.at[pl.ds(s * RPS, RPS)])
            for t in range(CHUNK // 16):
                ones_v[pl.ds(t * 16, 16)] = jnp.full((16,), 1.0, jnp.float32)
        plsc.subcore_barrier()

        pltpu.sync_copy(src_hbm.at[wid], src_v)
        pltpu.sync_copy(dst_hbm.at[wid], dst_v)

        def chunk_body(j, carry):
            pltpu.async_copy(h_hbm.at[src_v.at[j]], rows_v, sem).wait()
            pltpu.sync_copy(rows_v, accum.at[dst_v.at[j]], add=True)
            if with_deg:
                pltpu.sync_copy(ones_v, dacc.at[dst_v.at[j]], add=True)
            return carry

        lax.fori_loop(0, NCHUNK, chunk_body, 0)
        plsc.subcore_barrier()

        pltpu.sync_copy(accum.at[pl.ds(s * RPS, RPS)],
                        agg_out.at[pl.ds(c * N_PAD + s * RPS, RPS)])
        if with_deg:
            pltpu.sync_copy(dacc.at[pl.ds(s * RPS, RPS)],
                            deg_out.at[pl.ds(c * N_PAD + s * RPS, RPS)])


def _edge_agg(h, src_r, dst_r, zeros_pad, zeros1, with_deg):
    d = h.shape[1]
    out_type = [jax.ShapeDtypeStruct((NC * N_PAD, d), jnp.float32)]
    scratch = [
        pltpu.VMEM((NCHUNK, CHUNK), jnp.int32),
        pltpu.VMEM((NCHUNK, CHUNK), jnp.int32),
        pltpu.VMEM((CHUNK, d), jnp.float32),
    ]
    if with_deg:
        out_type.append(jax.ShapeDtypeStruct((NC * N_PAD,), jnp.float32))
        scratch.append(pltpu.VMEM((CHUNK,), jnp.float32))
    scratch.append(pltpu.VMEM_SHARED((N_PAD, d), jnp.float32))
    if with_deg:
        scratch.append(pltpu.VMEM_SHARED((N_PAD,), jnp.float32))
    scratch.append(pltpu.SemaphoreType.DMA)
    mesh = plsc.VectorSubcoreMesh(core_axis_name="c", subcore_axis_name="s")
    fn = pl.kernel(
        functools.partial(_edge_agg_body, with_deg, d),
        compiler_params=pltpu.CompilerParams(use_tc_tiling_on_sc=False),
        out_type=tuple(out_type),
        mesh=mesh,
        scratch_types=tuple(scratch),
    )
    return fn(h, src_r, dst_r, zeros_pad, zeros1)


def _combine0_body(p0, p1, h, w, b, d0, d1, h_out, degs_out):
    degs = d0[...] + d1[...] + 1.0
    degs_out[...] = degs
    agg = p0[...] + p1[...] + h[...]
    lin = jnp.dot(agg, w[...]) + b[...]
    h_out[...] = jnp.tanh(lin / degs)


def _combine_body(p0, p1, h, w, b, degs, h_out):
    agg = p0[...] + p1[...] + h[...]
    lin = jnp.dot(agg, w[...]) + b[...]
    h_out[...] = jnp.tanh(lin / degs[...])


def _combine0(p0, p1, h, w, b, d0, d1):
    return pl.pallas_call(
        _combine0_body,
        out_shape=(
            jax.ShapeDtypeStruct((N, w.shape[1]), jnp.float32),
            jax.ShapeDtypeStruct((N, 1), jnp.float32),
        ),
    )(p0, p1, h, w, b, d0, d1)


def _combine(p0, p1, h, w, b, degs):
    return pl.pallas_call(
        _combine_body,
        out_shape=jax.ShapeDtypeStruct((N, w.shape[1]), jnp.float32),
    )(p0, p1, h, w, b, degs)


def _head_body(p_ref, wc1_ref, bc1_ref, wc2_ref, bc2_ref, wd_ref, bd_ref, out_ref):
    p = p_ref[...]
    y1 = jnp.dot(p, wc1_ref[...], preferred_element_type=jnp.float32)
    y1 = jnp.maximum(y1 + bc1_ref[...], 0.0)
    yp = jnp.max(y1.reshape(G * (K // 2), 2, C1), axis=1)
    yp3 = yp.reshape(G, K // 2, C1)
    npos = K // 2 - KW2 + 1
    acc = jnp.zeros((G * npos, C2), jnp.float32)
    for dk in range(KW2):
        win = yp3[:, dk : dk + npos, :].reshape(G * npos, C1)
        acc = acc + jnp.dot(win, wc2_ref[dk], preferred_element_type=jnp.float32)
    y2 = jnp.maximum(acc + bc2_ref[...], 0.0)
    y23 = y2.reshape(G, npos, C2)
    oacc = jnp.zeros((G, OUT), jnp.float32)
    for j in range(npos):
        oacc = oacc + jnp.dot(
            y23[:, j, :], wd_ref[j * C2 : (j + 1) * C2, :],
            preferred_element_type=jnp.float32,
        )
    out_ref[...] = jnp.maximum(oacc + bd_ref[...], 0.0)


def _head(pooled_flat, Wc1_2d, bc1, Wc2_r, bc2, Wd_perm, bd):
    return pl.pallas_call(
        _head_body,
        out_shape=jax.ShapeDtypeStruct((G, OUT), jnp.float32),
    )(pooled_flat, Wc1_2d, bc1, Wc2_r, bc2, Wd_perm, bd)


def _topk_body(h_ref, out_ref):
    v = h_ref[...]
    lane = jax.lax.broadcasted_iota(jnp.int32, v.shape, 1)
    gbase = jax.lax.broadcasted_iota(jnp.int32, (G, 1), 0) * NPG
    neg = jnp.float32(-jnp.inf)
    for k in range(K):
        m = jnp.max(v, axis=1, keepdims=True)
        is_max = v >= m
        idx = jnp.min(jnp.where(is_max, lane, jnp.int32(2**30)), axis=1, keepdims=True)
        out_ref[:, k : k + 1] = idx + gbase
        v = jnp.where(lane == idx, neg, v)


def _topk_indices(h4):
    h4p = jnp.pad(h4, ((0, 0), (0, 128 - NPG)), constant_values=-jnp.inf)
    return pl.pallas_call(
        _topk_body,
        out_shape=jax.ShapeDtypeStruct((G, K), jnp.int32),
    )(h4p)


def kernel(node_feat, edge_index, W0, b0, W1, b1, W2, b2, W3, b3, Wc1, bc1, Wc2, bc2, Wd, bd):
    src = edge_index[0]
    dst = edge_index[1]
    bucket = dst // RPW
    onehot = (bucket[:, None] == jnp.arange(NW, dtype=jnp.int32)[None, :]).astype(jnp.int32)
    rank = jnp.take_along_axis(jnp.cumsum(onehot, axis=0), bucket[:, None], axis=1)[:, 0] - 1
    pos = bucket * EPW + rank
    fill_src = (jnp.arange(NW * EPW, dtype=jnp.int32) * 997) % N
    src_r = fill_src.at[pos].set(src, unique_indices=True).reshape(NW, NCHUNK, CHUNK)
    dst_r = jnp.full((NW * EPW,), N, jnp.int32).at[pos].set(dst, unique_indices=True).reshape(NW, NCHUNK, CHUNK)
    zeros128 = jnp.zeros((N_PAD, D), jnp.float32)
    zeros1 = jnp.zeros((N_PAD,), jnp.float32)

    h = node_feat
    degs = None
    cats = []
    for i, (W, b) in enumerate(((W0, b0), (W1, b1), (W2, b2), (W3, b3))):
        zp = zeros128[:, : h.shape[1]]
        if i == 0:
            aggp, degp = _edge_agg(h, src_r, dst_r, zp, zeros1, True)
            p0, p1 = aggp[:N], aggp[N_PAD : N_PAD + N]
            d0, d1 = degp[:N, None], degp[N_PAD : N_PAD + N, None]
            h, degs = _combine0(p0, p1, h, W, b, d0, d1)
        else:
            (aggp,) = _edge_agg(h, src_r, dst_r, zp, zeros1, False)
            p0, p1 = aggp[:N], aggp[N_PAD : N_PAD + N]
            h = _combine(p0, p1, h, W, b, degs)
        cats.append(h)

    cm = jnp.concatenate(cats, axis=1)
    idxg = _topk_indices(cats[3][:, 0].reshape(G, NPG))
    pooled = cm[idxg.reshape(-1)]
    Wc1_2d = Wc1[:, 0, :].T
    Wc2_r = jnp.transpose(Wc2, (2, 1, 0))
    Wd_perm = Wd.reshape(C2, 11, OUT).transpose(1, 0, 2).reshape(C2 * 11, OUT)
    out = _head(pooled, Wc1_2d, bc1, Wc2_r, bc2, Wd_perm, bd)
    return jnp.maximum(out, 0.0)
